# baseline (speedup 1.0000x reference)
import functools

import jax
import jax.numpy as jnp
from jax import lax
from jax.experimental import pallas as pl
from jax.experimental.pallas import tpu as pltpu
from jax.experimental.pallas import tpu_sc as plsc

_HIDDEN = 1024
_LANES = 16
_NCORES = 2
_NSUB = 16
_NW = _NCORES * _NSUB

_CHUNK = 16
_SC_FRAC_NUM, _SC_FRAC_DEN = 5, 8
_TC_TOK = 8


def _sc_body(ids_hbm, pids_hbm, word_hbm, ptab_hbm, out_hbm,
             idx_w, idx_p, w0, w1, p0, p1, o0, o1,
             sw0, sw1, sp0, sp1, so0, so1, *, per_w, nchunk):
    w = (w0, w1)
    p = (p0, p1)
    o = (o0, o1)
    sw = (sw0, sw1)
    sp = (sp0, sp1)
    so = (so0, so1)

    wid = lax.axis_index("s") * _NCORES + lax.axis_index("c")
    base = wid * per_w
    pltpu.sync_copy(ids_hbm.at[pl.ds(base, per_w)], idx_w)
    pltpu.sync_copy(pids_hbm.at[pl.ds(base, per_w)], idx_p)

    def gather_pair(ci, b):
        off = ci * _CHUNK
        pltpu.make_async_copy(word_hbm.at[idx_w.at[pl.ds(off, _CHUNK)]],
                              w[b], sw[b]).start()
        pltpu.make_async_copy(ptab_hbm.at[idx_p.at[pl.ds(off, _CHUNK)]],
                              p[b], sp[b]).start()

    def wait_gather(ci, b):
        off = ci * _CHUNK
        pltpu.make_async_copy(word_hbm.at[idx_w.at[pl.ds(off, _CHUNK)]],
                              w[b], sw[b]).wait()
        pltpu.make_async_copy(ptab_hbm.at[idx_p.at[pl.ds(off, _CHUNK)]],
                              p[b], sp[b]).wait()

    def start_out(ci, b):
        pltpu.make_async_copy(o[b], out_hbm.at[pl.ds(base + ci * _CHUNK,
                                                     _CHUNK)], so[b]).start()

    def wait_out(ci, b):
        pltpu.make_async_copy(o[b], out_hbm.at[pl.ds(base + ci * _CHUNK,
                                                     _CHUNK)], so[b]).wait()

    gather_pair(0, 0)

    def pair_body(g, carry):
        for b in (0, 1):
            ci = 2 * g + b
            if b == 0:
                gather_pair(ci + 1, 1 - b)
            else:
                @pl.when(g < (nchunk // 2) - 1)
                def _():
                    gather_pair(ci + 1, 1 - b)
            wait_gather(ci, b)

            @pl.when(g > 0)
            def _():
                wait_out(ci - 2, b)

            def add_row(r, c2):
                for j in range(_HIDDEN // _LANES):
                    sl = pl.ds(j * _LANES, _LANES)
                    o[b][r, sl] = w[b][r, sl] + p[b][r, sl]
                return c2

            lax.fori_loop(0, _CHUNK, add_row, 0, unroll=False)
            start_out(ci, b)
        return carry

    lax.fori_loop(0, nchunk // 2, pair_body, 0, unroll=False)
    wait_out(nchunk - 2, 0)
    wait_out(nchunk - 1, 1)


def _sc_call(ids, pids, word_table, pos_table):
    n = ids.shape[0]
    per_w = n // _NW
    nchunk = per_w // _CHUNK
    mesh = plsc.VectorSubcoreMesh(core_axis_name="c", subcore_axis_name="s")
    scratch = [pltpu.VMEM((per_w,), jnp.int32),
               pltpu.VMEM((per_w,), jnp.int32)]
    scratch += [pltpu.VMEM((_CHUNK, _HIDDEN), jnp.float32)
                for _ in range(6)]
    scratch += [pltpu.SemaphoreType.DMA for _ in range(6)]
    grid_kernel = pl.kernel(
        functools.partial(_sc_body, per_w=per_w, nchunk=nchunk),
        mesh=mesh,
        out_type=jax.ShapeDtypeStruct((n, _HIDDEN), jnp.float32),
        scratch_types=scratch,
    )
    return grid_kernel(ids, pids, word_table, pos_table)


def _tc_body(ids_ref, pids_ref, *refs):
    w_refs = refs[:_TC_TOK]
    p_refs = refs[_TC_TOK:2 * _TC_TOK]
    o_ref = refs[2 * _TC_TOK]
    for t in range(_TC_TOK):
        o_ref[t, :] = w_refs[t][0, 0, :] + p_refs[t][0, 0, :]


def _tc_call(ids, pids, word_table, pos_table):
    m = ids.shape[0]
    steps = m // _TC_TOK

    def w_map(t):
        return lambda i, ids_s, pids_s: (ids_s[i * _TC_TOK + t], 0, 0)

    def p_map(t):
        return lambda i, ids_s, pids_s: (pids_s[i * _TC_TOK + t], 0, 0)

    in_specs = [pl.BlockSpec((1, 1, _HIDDEN), w_map(t))
                for t in range(_TC_TOK)]
    in_specs += [pl.BlockSpec((1, 1, _HIDDEN), p_map(t))
                 for t in range(_TC_TOK)]
    grid_spec = pltpu.PrefetchScalarGridSpec(
        num_scalar_prefetch=2,
        grid=(steps,),
        in_specs=in_specs,
        out_specs=pl.BlockSpec((_TC_TOK, _HIDDEN),
                               lambda i, ids_s, pids_s: (i, 0)),
    )
    w3 = word_table.reshape(word_table.shape[0], 1, _HIDDEN)
    p3 = pos_table.reshape(pos_table.shape[0], 1, _HIDDEN)
    return pl.pallas_call(
        _tc_body,
        grid_spec=grid_spec,
        out_shape=jax.ShapeDtypeStruct((m, _HIDDEN), jnp.float32),
    )(ids, pids, *([w3] * _TC_TOK), *([p3] * _TC_TOK))


def kernel(input_ids, position_ids, word_table, pos_table):
    b, s = input_ids.shape
    n = b * s
    n_sc = (n * _SC_FRAC_NUM // _SC_FRAC_DEN) // (_NW * _CHUNK * 2) \
        * (_NW * _CHUNK * 2)
    n_tc = n - n_sc
    ids = input_ids.reshape(n).astype(jnp.int32)
    pids = position_ids.reshape(n).astype(jnp.int32)

    sc_out = _sc_call(ids[n_tc:], pids[n_tc:], word_table, pos_table)
    tc_out = _tc_call(ids[:n_tc], pids[:n_tc], word_table, pos_table)
    out = jnp.concatenate([tc_out, sc_out], axis=0)
    return out.reshape(b, s, _HIDDEN)


# speedup vs baseline: 9.8831x; 9.8831x over previous
import functools

import jax
import jax.numpy as jnp
from jax import lax
from jax.experimental import pallas as pl
from jax.experimental.pallas import tpu as pltpu
from jax.experimental.pallas import tpu_sc as plsc

_HIDDEN = 1024
_LANES = 16
_NCORES = 2
_NSUB = 16
_NW = _NCORES * _NSUB

_CHUNK = 16


def _emb_body(ids_hbm, pids_hbm, word_hbm, ptab_hbm, out_hbm,
              idx_w, idx_p, w0, w1, p0, p1, o0, o1,
              sw0, sw1, sp0, sp1, so0, so1, *, per_w, nchunk, wpb):
    w = (w0, w1)
    p = (p0, p1)
    o = (o0, o1)
    sw = (sw0, sw1)
    sp = (sp0, sp1)
    so = (so0, so1)

    wid = lax.axis_index("s") * _NCORES + lax.axis_index("c")
    bi = wid // wpb
    co = (wid % wpb) * per_w
    pltpu.sync_copy(ids_hbm.at[bi, pl.ds(co, per_w)], idx_w)
    pltpu.sync_copy(pids_hbm.at[bi, pl.ds(co, per_w)], idx_p)

    def gather_pair(ci, b):
        off = ci * _CHUNK
        pltpu.make_async_copy(word_hbm.at[idx_w.at[pl.ds(off, _CHUNK)]],
                              w[b], sw[b]).start()
        pltpu.make_async_copy(ptab_hbm.at[idx_p.at[pl.ds(off, _CHUNK)]],
                              p[b], sp[b]).start()

    def wait_gather(ci, b):
        off = ci * _CHUNK
        pltpu.make_async_copy(word_hbm.at[idx_w.at[pl.ds(off, _CHUNK)]],
                              w[b], sw[b]).wait()
        pltpu.make_async_copy(ptab_hbm.at[idx_p.at[pl.ds(off, _CHUNK)]],
                              p[b], sp[b]).wait()

    def start_out(ci, b):
        pltpu.make_async_copy(
            o[b], out_hbm.at[bi, pl.ds(co + ci * _CHUNK, _CHUNK)],
            so[b]).start()

    def wait_out(ci, b):
        pltpu.make_async_copy(
            o[b], out_hbm.at[bi, pl.ds(co + ci * _CHUNK, _CHUNK)],
            so[b]).wait()

    gather_pair(0, 0)

    def pair_body(g, carry):
        for b in (0, 1):
            ci = 2 * g + b
            if b == 0:
                gather_pair(ci + 1, 1 - b)
            else:
                @pl.when(g < (nchunk // 2) - 1)
                def _():
                    gather_pair(ci + 1, 1 - b)
            wait_gather(ci, b)

            @pl.when(g > 0)
            def _():
                wait_out(ci - 2, b)

            def add_row(r, c2):
                for j in range(_HIDDEN // _LANES):
                    sl = pl.ds(j * _LANES, _LANES)
                    o[b][r, sl] = w[b][r, sl] + p[b][r, sl]
                return c2

            lax.fori_loop(0, _CHUNK, add_row, 0, unroll=False)
            start_out(ci, b)
        return carry

    lax.fori_loop(0, nchunk // 2, pair_body, 0, unroll=False)
    wait_out(nchunk - 2, 0)
    wait_out(nchunk - 1, 1)


def kernel(input_ids, position_ids, word_table, pos_table):
    b, s = input_ids.shape
    n = b * s
    per_w = n // _NW
    nchunk = per_w // _CHUNK
    wpb = s // per_w
    if input_ids.dtype != jnp.int32:
        input_ids = input_ids.astype(jnp.int32)
    if position_ids.dtype != jnp.int32:
        position_ids = position_ids.astype(jnp.int32)

    mesh = plsc.VectorSubcoreMesh(core_axis_name="c", subcore_axis_name="s")
    scratch = [pltpu.VMEM((per_w,), jnp.int32),
               pltpu.VMEM((per_w,), jnp.int32)]
    scratch += [pltpu.VMEM((_CHUNK, _HIDDEN), jnp.float32)
                for _ in range(6)]
    scratch += [pltpu.SemaphoreType.DMA for _ in range(6)]
    grid_kernel = pl.kernel(
        functools.partial(_emb_body, per_w=per_w, nchunk=nchunk, wpb=wpb),
        mesh=mesh,
        out_type=jax.ShapeDtypeStruct((b, s, _HIDDEN), jnp.float32),
        scratch_types=scratch,
    )
    return grid_kernel(input_ids, position_ids, word_table, pos_table)
